# baseline (speedup 1.0000x reference)
import math

import jax
import jax.numpy as jnp
from jax.experimental import pallas as pl
from jax.experimental.pallas import tpu as pltpu

_HD = 128
_T = 4096
_B = 16
_PL0 = 256
_PN0 = 16
_PN1 = 4
_QL = 1024
_NQ = _T // _QL
_PPQ = _QL // _PL0
_NBUF = 8
_IR3 = 1.0 / math.sqrt(3.0)

_C0 = _IR3 * (2.0 * _IR3 + 0.5)
_C1 = 0.5 * (_IR3 + 1.0 + 0.5)

_D1 = (_IR3, 0.5, 0.5, _IR3)
_U1 = tuple(
    _D1[j] * ((_D1[j - 1] if j > 0 else 0.0) + 2.0 * _D1[j] + (_D1[j + 1] if j < 3 else 0.0)) / 4.0
    for j in range(4)
)


def _body(len_ref, x_ref, w0_ref, b0_ref, w1_ref, b1_ref, w2_ref, b2_ref,
          o_ref, buf_ref, s_ref, sstar_ref, sem_ref):

    def _nq_of(ln):
        return (ln + _QL - 1) // _QL

    def _copy(b, slot, k):
        return pltpu.make_async_copy(
            x_ref.at[b, :, pl.ds(0, k * _QL)],
            buf_ref.at[slot, :, pl.ds(0, k * _QL)],
            sem_ref.at[slot],
        )

    def _issue(b, slot):
        k = _nq_of(len_ref[b])
        for kk in range(1, _NQ + 1):
            @pl.when(k == kk)
            def _():
                _copy(b, slot, kk).start()

    def _wait(b, slot):
        k = _nq_of(len_ref[b])
        for kk in range(1, _NQ + 1):
            @pl.when(k == kk)
            def _():
                _copy(b, slot, kk).wait()

    for i in range(_NBUF - 1):
        @pl.when(len_ref[i] > 0)
        def _prologue(i=i):
            _issue(i, i)

    half = jnp.float32(0.5)
    ir3 = jnp.float32(_IR3)
    jq = jax.lax.broadcasted_iota(jnp.int32, (1, _QL), 1)
    jjq = jq % _PL0
    cvec = jnp.where(
        (jjq == 0) | (jjq == _PL0 - 1), jnp.float32(_C0),
        jnp.where((jjq == 1) | (jjq == _PL0 - 2), jnp.float32(_C1),
                  jnp.float32(1.0))) * jnp.float32(1.0 / _PL0)
    irow = jax.lax.broadcasted_iota(jnp.int32, (_QL, _PPQ), 0) // _PL0
    icol = jax.lax.broadcasted_iota(jnp.int32, (_QL, _PPQ), 1)
    ind = (irow == icol).astype(jnp.float32)
    jp = jax.lax.broadcasted_iota(jnp.int32, (1, _PL0), 1)
    ddp = jnp.where((jp == 0) | (jp == _PL0 - 1), ir3, half)
    dm1p = jnp.where(jp == 1, ir3, half)
    dp1p = jnp.where(jp == _PL0 - 2, ir3, half)

    def _step(b, carry):
        ln = len_ref[b]
        slot = b % _NBUF

        @pl.when(b + _NBUF - 1 < _B)
        def _issue_next():
            @pl.when(len_ref[b + _NBUF - 1] > 0)
            def _():
                _issue(b + _NBUF - 1, (b + _NBUF - 1) % _NBUF)

        _wait(b, slot)

        for q in range(_NQ):
            @pl.when(ln > q * _QL)
            def _pool_quarter(q=q):
                xq = buf_ref[slot][:, q * _QL:(q + 1) * _QL]
                s_ref[q] = jax.lax.dot_general(
                    xq * cvec, ind, (((1,), (0,)), ((), ())),
                    preferred_element_type=jnp.float32)

            @pl.when(ln <= q * _QL)
            def _zero_quarter(q=q):
                s_ref[q] = jnp.zeros((_HD, _PPQ), jnp.float32)

        pstar = ln // _PL0
        mstar = ln - pstar * _PL0
        plast = jnp.maximum((ln + _PL0 - 1) // _PL0 - 1, 0)
        psafe = jnp.minimum(jnp.minimum(pstar, plast), _PN0 - 1)

        @pl.when(ln > 0)
        def _pool_partial():
            base = pl.multiple_of(psafe * _PL0, _PL0)
            xs = buf_ref[slot, :, pl.ds(base, _PL0)]
            m = mstar
            gp = ((jp >= 1) & (jp <= m)).astype(jnp.float32)
            gs = (jp < m).astype(jnp.float32)
            gn = ((jp <= _PL0 - 2) & (jp + 1 < m)).astype(jnp.float32)
            w = ddp * (dm1p * gp + 2.0 * ddp * gs + dp1p * gn)
            w = w / jnp.maximum(m.astype(jnp.float32), 1.0)
            sstar_ref[...] = jnp.sum(xs * w, axis=1, keepdims=True)

        @pl.when(ln == 0)
        def _zero_partial():
            sstar_ref[...] = jnp.zeros((_HD, 1), jnp.float32)

        s_all = jnp.concatenate([s_ref[i] for i in range(_NQ)], axis=1)
        pidx = jax.lax.broadcasted_iota(jnp.int32, (1, _PN0), 1)
        s0 = (jnp.where(pidx < pstar, s_all, jnp.float32(0.0))
              + jnp.where(pidx == pstar, sstar_ref[...], jnp.float32(0.0)))

        h0 = jax.lax.dot_general(
            w0_ref[...], s0, (((0,), (0,)), ((), ())),
            preferred_element_type=jnp.float32)
        gate = (ln > pidx * _PL0).astype(jnp.float32)
        h0 = h0 + b0_ref[...] * gate

        cols1 = [
            _U1[0] * h0[:, 4 * u:4 * u + 1]
            + _U1[1] * h0[:, 4 * u + 1:4 * u + 2]
            + _U1[2] * h0[:, 4 * u + 2:4 * u + 3]
            + _U1[3] * h0[:, 4 * u + 3:4 * u + 4]
            for u in range(_PN1)
        ]
        s1 = jnp.concatenate(cols1, axis=1)

        h1 = jax.lax.dot_general(
            w1_ref[...], s1, (((0,), (0,)), ((), ())),
            preferred_element_type=jnp.float32) + b1_ref[...]
        out = jax.lax.dot_general(
            w2_ref[...], h1, (((0,), (0,)), ((), ())),
            preferred_element_type=jnp.float32) + b2_ref[...]
        o_ref[b] = out
        return carry

    jax.lax.fori_loop(0, _B, _step, 0)


def kernel(x, lengths, W0, b0, W1, b1, W2, b2):
    b0c = b0.reshape(_HD, 1)
    b1c = b1.reshape(_HD, 1)
    b2c = b2.reshape(_HD, 1)
    wspec = pl.BlockSpec((_HD, _HD), lambda i, L: (0, 0))
    bspec = pl.BlockSpec((_HD, 1), lambda i, L: (0, 0))
    return pl.pallas_call(
        _body,
        grid_spec=pltpu.PrefetchScalarGridSpec(
            num_scalar_prefetch=1,
            grid=(1,),
            in_specs=[
                pl.BlockSpec(memory_space=pltpu.MemorySpace.HBM),
                wspec, bspec, wspec, bspec, wspec, bspec,
            ],
            out_specs=pl.BlockSpec(
                (_B, _HD, _PN1), lambda i, L: (0, 0, 0)),
            scratch_shapes=[
                pltpu.VMEM((_NBUF, _HD, _T), jnp.float32),
                pltpu.VMEM((_NQ, _HD, _PPQ), jnp.float32),
                pltpu.VMEM((_HD, 1), jnp.float32),
                pltpu.SemaphoreType.DMA((_NBUF,)),
            ],
        ),
        out_shape=jax.ShapeDtypeStruct((_B, _HD, _PN1), jnp.float32),
    )(lengths, x, W0, b0c, W1, b1c, W2, b2c)


# speedup vs baseline: 1.0889x; 1.0889x over previous
import math

import jax
import jax.numpy as jnp
from jax.experimental import pallas as pl
from jax.experimental.pallas import tpu as pltpu

_HD = 128
_T = 4096
_B = 16
_PL0 = 256
_PN0 = 16
_PN1 = 4
_QL = 1024
_NQ = _T // _QL
_PPQ = _QL // _PL0
_NBUF = 8
_IR3 = 1.0 / math.sqrt(3.0)

_C0 = _IR3 * (2.0 * _IR3 + 0.5)
_C1 = 0.5 * (_IR3 + 1.0 + 0.5)

_D1 = (_IR3, 0.5, 0.5, _IR3)
_U1 = tuple(
    _D1[j] * ((_D1[j - 1] if j > 0 else 0.0) + 2.0 * _D1[j] + (_D1[j + 1] if j < 3 else 0.0)) / 4.0
    for j in range(4)
)


def _body(len_ref, x_ref, w0_ref, b0_ref, w1_ref, b1_ref, w2_ref, b2_ref,
          o_ref, buf_ref, s_ref, sstar_ref, sem_ref):

    def _nq_of(ln):
        return (ln + _QL - 1) // _QL

    def _copy(b, slot, k):
        return pltpu.make_async_copy(
            x_ref.at[b, :, pl.ds(0, k * _QL)],
            buf_ref.at[slot, :, pl.ds(0, k * _QL)],
            sem_ref.at[slot],
        )

    def _issue(b, slot):
        k = _nq_of(len_ref[b])
        for kk in range(1, _NQ + 1):
            @pl.when(k == kk)
            def _():
                _copy(b, slot, kk).start()

    def _wait(b, slot):
        k = _nq_of(len_ref[b])
        for kk in range(1, _NQ + 1):
            @pl.when(k == kk)
            def _():
                _copy(b, slot, kk).wait()

    for i in range(_NBUF - 1):
        @pl.when(len_ref[i] > 0)
        def _prologue(i=i):
            _issue(i, i)

    half = jnp.float32(0.5)
    ir3 = jnp.float32(_IR3)
    jq = jax.lax.broadcasted_iota(jnp.int32, (1, _QL), 1)
    jjq = jq % _PL0
    cvec = jnp.where(
        (jjq == 0) | (jjq == _PL0 - 1), jnp.float32(_C0),
        jnp.where((jjq == 1) | (jjq == _PL0 - 2), jnp.float32(_C1),
                  jnp.float32(1.0))) * jnp.float32(1.0 / _PL0)
    jp = jax.lax.broadcasted_iota(jnp.int32, (1, _PL0), 1)
    ddp = jnp.where((jp == 0) | (jp == _PL0 - 1), ir3, half)
    dm1p = jnp.where(jp == 1, ir3, half)
    dp1p = jnp.where(jp == _PL0 - 2, ir3, half)

    def _step(b, carry):
        ln = len_ref[b]
        slot = b % _NBUF

        @pl.when(b + _NBUF - 1 < _B)
        def _issue_next():
            @pl.when(len_ref[b + _NBUF - 1] > 0)
            def _():
                _issue(b + _NBUF - 1, (b + _NBUF - 1) % _NBUF)

        _wait(b, slot)

        for q in range(_NQ):
            @pl.when(ln > q * _QL)
            def _pool_quarter(q=q):
                xq = buf_ref[slot][:, q * _QL:(q + 1) * _QL]
                xc = xq * cvec
                cols = [
                    jnp.sum(xc[:, kk * _PL0:(kk + 1) * _PL0], axis=1, keepdims=True)
                    for kk in range(_PPQ)
                ]
                s_ref[q] = jnp.concatenate(cols, axis=1)

            @pl.when(ln <= q * _QL)
            def _zero_quarter(q=q):
                s_ref[q] = jnp.zeros((_HD, _PPQ), jnp.float32)

        pstar = ln // _PL0
        mstar = ln - pstar * _PL0
        plast = jnp.maximum((ln + _PL0 - 1) // _PL0 - 1, 0)
        psafe = jnp.minimum(jnp.minimum(pstar, plast), _PN0 - 1)

        @pl.when(ln > 0)
        def _pool_partial():
            base = pl.multiple_of(psafe * _PL0, _PL0)
            xs = buf_ref[slot, :, pl.ds(base, _PL0)]
            m = mstar
            gp = ((jp >= 1) & (jp <= m)).astype(jnp.float32)
            gs = (jp < m).astype(jnp.float32)
            gn = ((jp <= _PL0 - 2) & (jp + 1 < m)).astype(jnp.float32)
            w = ddp * (dm1p * gp + 2.0 * ddp * gs + dp1p * gn)
            w = w / jnp.maximum(m.astype(jnp.float32), 1.0)
            sstar_ref[...] = jnp.sum(xs * w, axis=1, keepdims=True)

        @pl.when(ln == 0)
        def _zero_partial():
            sstar_ref[...] = jnp.zeros((_HD, 1), jnp.float32)

        s_all = jnp.concatenate([s_ref[i] for i in range(_NQ)], axis=1)
        pidx = jax.lax.broadcasted_iota(jnp.int32, (1, _PN0), 1)
        s0 = (jnp.where(pidx < pstar, s_all, jnp.float32(0.0))
              + jnp.where(pidx == pstar, sstar_ref[...], jnp.float32(0.0)))

        h0 = jax.lax.dot_general(
            w0_ref[...], s0, (((0,), (0,)), ((), ())),
            preferred_element_type=jnp.float32)
        gate = (ln > pidx * _PL0).astype(jnp.float32)
        h0 = h0 + b0_ref[...] * gate

        cols1 = [
            _U1[0] * h0[:, 4 * u:4 * u + 1]
            + _U1[1] * h0[:, 4 * u + 1:4 * u + 2]
            + _U1[2] * h0[:, 4 * u + 2:4 * u + 3]
            + _U1[3] * h0[:, 4 * u + 3:4 * u + 4]
            for u in range(_PN1)
        ]
        s1 = jnp.concatenate(cols1, axis=1)

        h1 = jax.lax.dot_general(
            w1_ref[...], s1, (((0,), (0,)), ((), ())),
            preferred_element_type=jnp.float32) + b1_ref[...]
        out = jax.lax.dot_general(
            w2_ref[...], h1, (((0,), (0,)), ((), ())),
            preferred_element_type=jnp.float32) + b2_ref[...]
        o_ref[b] = out
        return carry

    jax.lax.fori_loop(0, _B, _step, 0)


def kernel(x, lengths, W0, b0, W1, b1, W2, b2):
    b0c = b0.reshape(_HD, 1)
    b1c = b1.reshape(_HD, 1)
    b2c = b2.reshape(_HD, 1)
    wspec = pl.BlockSpec((_HD, _HD), lambda i, L: (0, 0))
    bspec = pl.BlockSpec((_HD, 1), lambda i, L: (0, 0))
    return pl.pallas_call(
        _body,
        grid_spec=pltpu.PrefetchScalarGridSpec(
            num_scalar_prefetch=1,
            grid=(1,),
            in_specs=[
                pl.BlockSpec(memory_space=pltpu.MemorySpace.HBM),
                wspec, bspec, wspec, bspec, wspec, bspec,
            ],
            out_specs=pl.BlockSpec(
                (_B, _HD, _PN1), lambda i, L: (0, 0, 0)),
            scratch_shapes=[
                pltpu.VMEM((_NBUF, _HD, _T), jnp.float32),
                pltpu.VMEM((_NQ, _HD, _PPQ), jnp.float32),
                pltpu.VMEM((_HD, 1), jnp.float32),
                pltpu.SemaphoreType.DMA((_NBUF,)),
            ],
        ),
        out_shape=jax.ShapeDtypeStruct((_B, _HD, _PN1), jnp.float32),
    )(lengths, x, W0, b0c, W1, b1c, W2, b2c)


# speedup vs baseline: 1.3083x; 1.2014x over previous
import math

import jax
import jax.numpy as jnp
from jax.experimental import pallas as pl
from jax.experimental.pallas import tpu as pltpu

_HD = 128
_T = 4096
_B = 16
_PL0 = 256
_PN0 = 16
_PN1 = 4
_QL = 1024
_NQ = _T // _QL
_PPQ = _QL // _PL0
_NBUF = 8
_IR3 = 1.0 / math.sqrt(3.0)

_C0 = _IR3 * (2.0 * _IR3 + 0.5)
_C1 = 0.5 * (_IR3 + 1.0 + 0.5)

_D1 = (_IR3, 0.5, 0.5, _IR3)
_U1 = tuple(
    _D1[j] * ((_D1[j - 1] if j > 0 else 0.0) + 2.0 * _D1[j] + (_D1[j + 1] if j < 3 else 0.0)) / 4.0
    for j in range(4)
)


def _body(len_ref, x_ref, w0_ref, b0_ref, w1_ref, b1_ref, w2_ref, b2_ref,
          o_ref, buf_ref, sem_ref):

    def _nq_of(ln):
        return (ln + _QL - 1) // _QL

    def _copy(b, slot, k):
        return pltpu.make_async_copy(
            x_ref.at[b, :, pl.ds(0, k * _QL)],
            buf_ref.at[slot, :, pl.ds(0, k * _QL)],
            sem_ref.at[slot],
        )

    def _issue(b, slot):
        k = _nq_of(len_ref[b])
        for kk in range(1, _NQ + 1):
            @pl.when(k == kk)
            def _():
                _copy(b, slot, kk).start()

    for i in range(_NBUF - 1):
        @pl.when(len_ref[i] > 0)
        def _prologue(i=i):
            _issue(i, i)

    half = jnp.float32(0.5)
    ir3 = jnp.float32(_IR3)
    jq = jax.lax.broadcasted_iota(jnp.int32, (1, _QL), 1)
    jjq = jq % _PL0
    cvec = jnp.where(
        (jjq == 0) | (jjq == _PL0 - 1), jnp.float32(_C0),
        jnp.where((jjq == 1) | (jjq == _PL0 - 2), jnp.float32(_C1),
                  jnp.float32(1.0))) * jnp.float32(1.0 / _PL0)
    jp = jax.lax.broadcasted_iota(jnp.int32, (1, _PL0), 1)
    ddp = jnp.where((jp == 0) | (jp == _PL0 - 1), ir3, half)
    dm1p = jnp.where(jp == 1, ir3, half)
    dp1p = jnp.where(jp == _PL0 - 2, ir3, half)
    pidx = jax.lax.broadcasted_iota(jnp.int32, (1, _PN0), 1)

    def _finish(b, ln, s0):
        h0 = jax.lax.dot_general(
            w0_ref[...], s0, (((0,), (0,)), ((), ())),
            preferred_element_type=jnp.float32)
        gate = (ln > pidx * _PL0).astype(jnp.float32)
        h0 = h0 + b0_ref[...] * gate
        cols1 = [
            _U1[0] * h0[:, 4 * u:4 * u + 1]
            + _U1[1] * h0[:, 4 * u + 1:4 * u + 2]
            + _U1[2] * h0[:, 4 * u + 2:4 * u + 3]
            + _U1[3] * h0[:, 4 * u + 3:4 * u + 4]
            for u in range(_PN1)
        ]
        s1 = jnp.concatenate(cols1, axis=1)
        h1 = jax.lax.dot_general(
            w1_ref[...], s1, (((0,), (0,)), ((), ())),
            preferred_element_type=jnp.float32) + b1_ref[...]
        out = jax.lax.dot_general(
            w2_ref[...], h1, (((0,), (0,)), ((), ())),
            preferred_element_type=jnp.float32) + b2_ref[...]
        o_ref[b] = out

    def _do_batch(b, ln, slot, kk):
        cols = []
        for q in range(kk):
            xq = buf_ref[slot][:, q * _QL:(q + 1) * _QL]
            xc = xq * cvec
            cols.extend(
                jnp.sum(xc[:, t * _PL0:(t + 1) * _PL0], axis=1, keepdims=True)
                for t in range(_PPQ))
        if kk < _NQ:
            cols.append(jnp.zeros((_HD, _PPQ * (_NQ - kk)), jnp.float32))
        s_all = jnp.concatenate(cols, axis=1)

        pstar = ln // _PL0
        mstar = ln - pstar * _PL0
        psafe = jnp.minimum(pstar, kk * _PPQ - 1)
        base = pl.multiple_of(psafe * _PL0, _PL0)
        xs = buf_ref[slot, :, pl.ds(base, _PL0)]
        gp = ((jp >= 1) & (jp <= mstar)).astype(jnp.float32)
        gs = (jp < mstar).astype(jnp.float32)
        gn = ((jp <= _PL0 - 2) & (jp + 1 < mstar)).astype(jnp.float32)
        w = ddp * (dm1p * gp + 2.0 * ddp * gs + dp1p * gn)
        w = w / jnp.maximum(mstar.astype(jnp.float32), 1.0)
        sstar = jnp.sum(xs * w, axis=1, keepdims=True)

        s0 = (jnp.where(pidx < pstar, s_all, jnp.float32(0.0))
              + jnp.where(pidx == pstar, sstar, jnp.float32(0.0)))
        _finish(b, ln, s0)

    def _step(b, carry):
        ln = len_ref[b]
        slot = b % _NBUF

        @pl.when(b + _NBUF - 1 < _B)
        def _issue_next():
            @pl.when(len_ref[b + _NBUF - 1] > 0)
            def _():
                _issue(b + _NBUF - 1, (b + _NBUF - 1) % _NBUF)

        k = _nq_of(ln)

        @pl.when(k == 0)
        def _empty_batch():
            _finish(b, ln, jnp.zeros((_HD, _PN0), jnp.float32))

        for kk in range(1, _NQ + 1):
            @pl.when(k == kk)
            def _valid_batch(kk=kk):
                _copy(b, slot, kk).wait()
                _do_batch(b, ln, slot, kk)

        return carry

    jax.lax.fori_loop(0, _B, _step, 0)


def kernel(x, lengths, W0, b0, W1, b1, W2, b2):
    b0c = b0.reshape(_HD, 1)
    b1c = b1.reshape(_HD, 1)
    b2c = b2.reshape(_HD, 1)
    wspec = pl.BlockSpec((_HD, _HD), lambda i, L: (0, 0))
    bspec = pl.BlockSpec((_HD, 1), lambda i, L: (0, 0))
    return pl.pallas_call(
        _body,
        grid_spec=pltpu.PrefetchScalarGridSpec(
            num_scalar_prefetch=1,
            grid=(1,),
            in_specs=[
                pl.BlockSpec(memory_space=pltpu.MemorySpace.HBM),
                wspec, bspec, wspec, bspec, wspec, bspec,
            ],
            out_specs=pl.BlockSpec(
                (_B, _HD, _PN1), lambda i, L: (0, 0, 0)),
            scratch_shapes=[
                pltpu.VMEM((_NBUF, _HD, _T), jnp.float32),
                pltpu.SemaphoreType.DMA((_NBUF,)),
            ],
        ),
        out_shape=jax.ShapeDtypeStruct((_B, _HD, _PN1), jnp.float32),
    )(lengths, x, W0, b0c, W1, b1c, W2, b2c)
